# baseline (speedup 1.0000x reference)
import functools

import jax
import jax.numpy as jnp
from jax import lax
from jax.experimental import pallas as pl
from jax.experimental.pallas import tpu as pltpu
from jax.experimental.pallas import tpu_sc as plsc

EXPERTS = 64
TOPK = 8
TOKENS = 32768
DMODEL = 768
NC, NS = 2, 16
NW = NC * NS
TOK_PER_W = TOKENS // NW
GROUPS = TOK_PER_W // 16
MM_TILE = 8192

_SORT8 = ((0, 1), (2, 3), (4, 5), (6, 7), (0, 2), (1, 3), (4, 6), (5, 7),
          (1, 2), (5, 6), (0, 4), (1, 5), (2, 6), (3, 7), (2, 4), (3, 5),
          (1, 2), (3, 4), (5, 6))
_BITONIC8 = ((0, 4), (1, 5), (2, 6), (3, 7), (0, 2), (1, 3), (4, 6), (5, 7),
             (0, 1), (2, 3), (4, 5), (6, 7))


def _logits_body(w_ref, h_ref, out_ref):
    out_ref[...] = lax.dot_general(
        h_ref[...], w_ref[...], (((1,), (1,)), ((), ())),
        preferred_element_type=jnp.float32)


def _logits_t(hf, W):
    return pl.pallas_call(
        _logits_body,
        grid=(TOKENS // MM_TILE,),
        in_specs=[
            pl.BlockSpec((EXPERTS, DMODEL), lambda i: (0, 0)),
            pl.BlockSpec((MM_TILE, DMODEL), lambda i: (i, 0)),
        ],
        out_specs=pl.BlockSpec((MM_TILE, EXPERTS), lambda i: (i, 0)),
        out_shape=jax.ShapeDtypeStruct((TOKENS, EXPERTS), jnp.float32),
    )(W, hf)


def _cswap(v, ix, a, b):
    cnd = v[b] > v[a]
    v[a], v[b] = jnp.where(cnd, v[b], v[a]), jnp.where(cnd, v[a], v[b])
    ix[a], ix[b] = jnp.where(cnd, ix[b], ix[a]), jnp.where(cnd, ix[a], ix[b])


def _topk_tec(lgT, ids_out, w_out, lg_v, ids_v, w_v):
    c = lax.axis_index("c")
    s = lax.axis_index("s")
    wid = s * NC + c
    base = wid * TOK_PER_W
    pltpu.sync_copy(lgT.at[pl.ds(wid * (TOK_PER_W // 2), TOK_PER_W // 2), :], lg_v)
    lanes = lax.iota(jnp.int32, 16)
    lane8 = lanes * TOPK
    colb = (lanes & 1) * 64

    def group(g, _):
        g16 = g * 16
        vs = [jnp.full((16,), -jnp.inf, jnp.float32) for _ in range(TOPK)]
        ix = [jnp.zeros((16,), jnp.int32) for _ in range(TOPK)]
        rows = (lanes + g16) >> 1
        for cidx in range(EXPERTS // 8):
            v = [plsc.load_gather(lg_v, [rows, colb + (cidx * 8 + u)])
                 for u in range(8)]
            vi = [jnp.full((16,), cidx * 8 + u, jnp.int32) for u in range(8)]
            for a, b in _SORT8:
                _cswap(v, vi, a, b)
            for j in range(TOPK):
                cnd = v[7 - j] > vs[j]
                vs[j] = jnp.where(cnd, v[7 - j], vs[j])
                ix[j] = jnp.where(cnd, vi[7 - j], ix[j])
            for a, b in _BITONIC8:
                _cswap(vs, ix, a, b)
        es = [jnp.exp(t - vs[0]) for t in vs]
        tot = es[0]
        for t in es[1:]:
            tot = tot + t
        pos = lane8 + g * (16 * TOPK)
        for j in range(TOPK):
            plsc.store_scatter(ids_v, [pos + j], ix[j])
            plsc.store_scatter(w_v, [pos + j], es[j] / tot)
        return 0

    lax.fori_loop(0, GROUPS, group, 0)
    pltpu.sync_copy(ids_v, ids_out.at[pl.ds(base * TOPK, TOK_PER_W * TOPK)])
    pltpu.sync_copy(w_v, w_out.at[pl.ds(base * TOPK, TOK_PER_W * TOPK)])


_topk_call = pl.kernel(
    _topk_tec,
    out_type=[
        jax.ShapeDtypeStruct((TOKENS * TOPK,), jnp.int32),
        jax.ShapeDtypeStruct((TOKENS * TOPK,), jnp.float32),
    ],
    mesh=plsc.VectorSubcoreMesh(
        core_axis_name="c", subcore_axis_name="s",
        num_cores=NC, num_subcores=NS),
    compiler_params=pltpu.CompilerParams(needs_layout_passes=False),
    scratch_types=[
        pltpu.VMEM((TOK_PER_W // 2, 128), jnp.float32),
        pltpu.VMEM((TOK_PER_W * TOPK,), jnp.int32),
        pltpu.VMEM((TOK_PER_W * TOPK,), jnp.float32),
    ],
)


def kernel(h, W):
    hf = h.reshape(TOKENS, DMODEL)
    lg = _logits_t(hf, W).reshape(TOKENS // 2, 2 * EXPERTS)
    ids_flat, w_flat = _topk_call(lg)
    return (ids_flat.reshape(TOKENS, TOPK),
            w_flat.reshape(TOKENS, TOPK),
            jnp.float32(0.0))


# speedup vs baseline: 1.2732x; 1.2732x over previous
import functools

import jax
import jax.numpy as jnp
from jax import lax
from jax.experimental import pallas as pl
from jax.experimental.pallas import tpu as pltpu
from jax.experimental.pallas import tpu_sc as plsc

EXPERTS = 64
TOPK = 8
TOKENS = 32768
DMODEL = 768
NC, NS = 2, 16
NW = NC * NS
TOK_PER_W = TOKENS // NW
GROUPS = TOK_PER_W // 16
MM_TILE = 4096
OUT_ROWS = TOKENS * TOPK // 128
W_ROWS = TOK_PER_W * TOPK // 128


def _logits_body(w_ref, h_ref, out_ref):
    acc = lax.dot_general(
        h_ref[...], w_ref[...], (((1,), (1,)), ((), ())),
        preferred_element_type=jnp.float32)
    out_ref[...] = acc.T


def _logits_t(hf, W):
    return pl.pallas_call(
        _logits_body,
        grid=(TOKENS // MM_TILE,),
        in_specs=[
            pl.BlockSpec((EXPERTS, DMODEL), lambda i: (0, 0)),
            pl.BlockSpec((MM_TILE, DMODEL), lambda i: (i, 0)),
        ],
        out_specs=pl.BlockSpec((EXPERTS, MM_TILE), lambda i: (0, i)),
        out_shape=jax.ShapeDtypeStruct((EXPERTS, TOKENS), jnp.float32),
    )(W, hf)


def _topk_tec(lgT, ids_out, w_out, lg_v, ids_v, w_v):
    c = lax.axis_index("c")
    s = lax.axis_index("s")
    wid = s * NC + c
    base = wid * TOK_PER_W
    pltpu.sync_copy(lgT.at[:, pl.ds(base, TOK_PER_W)], lg_v)
    lanes = lax.iota(jnp.int32, 16)
    lane8 = lanes * TOPK
    one = jnp.full((16,), 1, jnp.int32)

    def group(g, _):
        g16 = g * 16

        def expert(e):
            return lg_v[e, pl.ds(g16, 16)], one * e

        vs = [None] * TOPK
        ix = [None] * TOPK
        vs[0], ix[0] = expert(0)
        for e in range(1, TOPK):
            x, xi = expert(e)
            for j in range(e):
                cnd = x > vs[j]
                vs[j], x = jnp.where(cnd, x, vs[j]), jnp.where(cnd, vs[j], x)
                ix[j], xi = jnp.where(cnd, xi, ix[j]), jnp.where(cnd, ix[j], xi)
            vs[e], ix[e] = x, xi
        for e in range(TOPK, EXPERTS):
            x, xi = expert(e)
            for j in range(TOPK):
                cnd = x > vs[j]
                vs[j], x = jnp.where(cnd, x, vs[j]), jnp.where(cnd, vs[j], x)
                ix[j], xi = jnp.where(cnd, xi, ix[j]), jnp.where(cnd, ix[j], xi)
        es = [jnp.exp(t - vs[0]) for t in vs]
        tot = es[0]
        for t in es[1:]:
            tot = tot + t
        rows = one * g
        for j in range(TOPK):
            plsc.store_scatter(ids_v, [rows, lane8 + j], ix[j])
            plsc.store_scatter(w_v, [rows, lane8 + j], es[j] / tot)
        return 0

    lax.fori_loop(0, GROUPS, group, 0)
    pltpu.sync_copy(ids_v, ids_out.at[pl.ds(wid * W_ROWS, W_ROWS), :])
    pltpu.sync_copy(w_v, w_out.at[pl.ds(wid * W_ROWS, W_ROWS), :])


_topk_call = pl.kernel(
    _topk_tec,
    out_type=[
        jax.ShapeDtypeStruct((OUT_ROWS, 128), jnp.int32),
        jax.ShapeDtypeStruct((OUT_ROWS, 128), jnp.float32),
    ],
    mesh=plsc.VectorSubcoreMesh(
        core_axis_name="c", subcore_axis_name="s",
        num_cores=NC, num_subcores=NS),
    compiler_params=pltpu.CompilerParams(needs_layout_passes=False),
    scratch_types=[
        pltpu.VMEM((EXPERTS, TOK_PER_W), jnp.float32),
        pltpu.VMEM((W_ROWS, 128), jnp.int32),
        pltpu.VMEM((W_ROWS, 128), jnp.float32),
    ],
)


def kernel(h, W):
    hf = h.reshape(TOKENS, DMODEL)
    lgT = _logits_t(hf, W)
    ids2d, w2d = _topk_call(lgT)
    return (ids2d.reshape(TOKENS, TOPK),
            w2d.reshape(TOKENS, TOPK),
            jnp.float32(0.0))


# speedup vs baseline: 1.3182x; 1.0353x over previous
import functools

import jax
import jax.numpy as jnp
from jax import lax
from jax.experimental import pallas as pl
from jax.experimental.pallas import tpu as pltpu
from jax.experimental.pallas import tpu_sc as plsc

EXPERTS = 64
TOPK = 8
TOKENS = 32768
DMODEL = 768
NC, NS = 2, 16
NW = NC * NS
TOK_PER_W = TOKENS // NW
GROUPS = TOK_PER_W // 16
MM_TILE = 4096
OUT_ROWS = TOKENS * TOPK // 128
W_ROWS = TOK_PER_W * TOPK // 128


def _logits_body(w_ref, h_ref, out_ref):
    acc = lax.dot_general(
        h_ref[...], w_ref[...], (((1,), (1,)), ((), ())),
        preferred_element_type=jnp.float32)
    out_ref[...] = acc.T


def _logits_t(hf, W):
    return pl.pallas_call(
        _logits_body,
        grid=(TOKENS // MM_TILE,),
        in_specs=[
            pl.BlockSpec((EXPERTS, DMODEL), lambda i: (0, 0)),
            pl.BlockSpec((MM_TILE, DMODEL), lambda i: (i, 0)),
        ],
        out_specs=pl.BlockSpec((EXPERTS, MM_TILE), lambda i: (0, i)),
        out_shape=jax.ShapeDtypeStruct((EXPERTS, TOKENS), jnp.float32),
    )(W, hf)


def _topk_tec(lgT, ids_out, w_out, lg_v, ids_v, w_v):
    c = lax.axis_index("c")
    s = lax.axis_index("s")
    wid = s * NC + c
    base = wid * TOK_PER_W
    pltpu.sync_copy(lgT.at[:, pl.ds(base, TOK_PER_W)], lg_v)
    lanes = lax.iota(jnp.int32, 16)
    lane8 = lanes * TOPK
    one = jnp.full((16,), 1, jnp.int32)

    def group(g, _):
        g16 = g * 16

        def expert(e):
            return lg_v[e, pl.ds(g16, 16)], one * e

        vs = [None] * TOPK
        ix = [None] * TOPK
        vs[0], ix[0] = expert(0)
        for e in range(1, TOPK):
            x, xi = expert(e)
            for j in range(e):
                cnd = x > vs[j]
                vs[j], x = jnp.where(cnd, x, vs[j]), jnp.where(cnd, vs[j], x)
                ix[j], xi = jnp.where(cnd, xi, ix[j]), jnp.where(cnd, ix[j], xi)
            vs[e], ix[e] = x, xi
        for e in range(TOPK, EXPERTS):
            x, xi = expert(e)
            for j in range(TOPK):
                cnd = x > vs[j]
                vs[j], x = jnp.where(cnd, x, vs[j]), jnp.where(cnd, vs[j], x)
                ix[j], xi = jnp.where(cnd, xi, ix[j]), jnp.where(cnd, ix[j], xi)
        es = [jnp.exp(t - vs[0]) for t in vs]
        tot = es[0]
        for t in es[1:]:
            tot = tot + t
        for j in range(TOPK):
            ids_v[j, pl.ds(g16, 16)] = ix[j]
            w_v[j, pl.ds(g16, 16)] = es[j] / tot
        return 0

    lax.fori_loop(0, GROUPS, group, 0)
    pltpu.sync_copy(ids_v, ids_out.at[:, pl.ds(base, TOK_PER_W)])
    pltpu.sync_copy(w_v, w_out.at[:, pl.ds(base, TOK_PER_W)])


_topk_call = pl.kernel(
    _topk_tec,
    out_type=[
        jax.ShapeDtypeStruct((TOPK, TOKENS), jnp.int32),
        jax.ShapeDtypeStruct((TOPK, TOKENS), jnp.float32),
    ],
    mesh=plsc.VectorSubcoreMesh(
        core_axis_name="c", subcore_axis_name="s",
        num_cores=NC, num_subcores=NS),
    compiler_params=pltpu.CompilerParams(needs_layout_passes=False),
    scratch_types=[
        pltpu.VMEM((EXPERTS, TOK_PER_W), jnp.float32),
        pltpu.VMEM((TOPK, TOK_PER_W), jnp.int32),
        pltpu.VMEM((TOPK, TOK_PER_W), jnp.float32),
    ],
)


def _unpack_body(ids_ref, w_ref, ids_out, w_out):
    ids_out[...] = ids_ref[...].T
    w_out[...] = w_ref[...].T


def _unpack(ids_sm, w_sm):
    blk = TOKENS // 16
    return pl.pallas_call(
        _unpack_body,
        grid=(16,),
        in_specs=[
            pl.BlockSpec((TOPK, blk), lambda i: (0, i)),
            pl.BlockSpec((TOPK, blk), lambda i: (0, i)),
        ],
        out_specs=[
            pl.BlockSpec((blk, TOPK), lambda i: (i, 0)),
            pl.BlockSpec((blk, TOPK), lambda i: (i, 0)),
        ],
        out_shape=[
            jax.ShapeDtypeStruct((TOKENS, TOPK), jnp.int32),
            jax.ShapeDtypeStruct((TOKENS, TOPK), jnp.float32),
        ],
    )(ids_sm, w_sm)


def kernel(h, W):
    hf = h.reshape(TOKENS, DMODEL)
    lgT = _logits_t(hf, W)
    ids_sm, w_sm = _topk_call(lgT)
    ids, w = _unpack(ids_sm, w_sm)
    return (ids, w, jnp.float32(0.0))


# speedup vs baseline: 1.8831x; 1.4286x over previous
import functools

import jax
import jax.numpy as jnp
from jax import lax
from jax.experimental import pallas as pl
from jax.experimental.pallas import tpu as pltpu
from jax.experimental.pallas import tpu_sc as plsc

EXPERTS = 64
TOPK = 8
TOKENS = 32768
DMODEL = 768
NC, NS = 2, 16
NW = NC * NS
TOK_PER_W = TOKENS // NW
GROUPS = TOK_PER_W // 16
MM_TILE = 4096
OUT_ROWS = TOKENS * TOPK // 128
W_ROWS = TOK_PER_W * TOPK // 128


def _logits_body(w_ref, h_ref, out_ref):
    acc = lax.dot_general(
        h_ref[...], w_ref[...], (((1,), (1,)), ((), ())),
        preferred_element_type=jnp.float32)
    out_ref[...] = acc.T


def _logits_t(hf, W):
    return pl.pallas_call(
        _logits_body,
        grid=(TOKENS // MM_TILE,),
        in_specs=[
            pl.BlockSpec((EXPERTS, DMODEL), lambda i: (0, 0)),
            pl.BlockSpec((MM_TILE, DMODEL), lambda i: (i, 0)),
        ],
        out_specs=pl.BlockSpec((EXPERTS, MM_TILE), lambda i: (0, i)),
        out_shape=jax.ShapeDtypeStruct((EXPERTS, TOKENS), jnp.float32),
    )(W, hf)


def _topk_tec(lgT, ids_out, w_out, lg_v, ids_v, w_v):
    c = lax.axis_index("c")
    s = lax.axis_index("s")
    wid = s * NC + c
    base = wid * TOK_PER_W
    pltpu.sync_copy(lgT.at[:, pl.ds(base, TOK_PER_W)], lg_v)
    lanes = lax.iota(jnp.int32, 16)
    lane8 = lanes * TOPK
    one = jnp.full((16,), 1, jnp.int32)

    def group(g, _):
        g16 = g * 16

        def expert(e):
            return lg_v[e, pl.ds(g16, 16)], one * e

        vs = [None] * TOPK
        ix = [None] * TOPK
        vs[0], ix[0] = expert(0)
        for e in range(1, TOPK):
            x, xi = expert(e)
            for j in range(e):
                cnd = x > vs[j]
                vs[j], x = jnp.where(cnd, x, vs[j]), jnp.where(cnd, vs[j], x)
                ix[j], xi = jnp.where(cnd, xi, ix[j]), jnp.where(cnd, ix[j], xi)
            vs[e], ix[e] = x, xi
        for e in range(TOPK, EXPERTS):
            x, xi = expert(e)
            for j in range(TOPK):
                cnd = x > vs[j]
                vs[j], x = jnp.where(cnd, x, vs[j]), jnp.where(cnd, vs[j], x)
                ix[j], xi = jnp.where(cnd, xi, ix[j]), jnp.where(cnd, ix[j], xi)
        es = [jnp.exp(t - vs[0]) for t in vs]
        tot = es[0]
        for t in es[1:]:
            tot = tot + t
        for j in range(TOPK):
            ids_v[j, pl.ds(g16, 16)] = ix[j]
            w_v[j, pl.ds(g16, 16)] = es[j] / tot
        return 0

    lax.fori_loop(0, GROUPS, group, 0)
    pltpu.sync_copy(ids_v, ids_out.at[:, pl.ds(base, TOK_PER_W)])
    pltpu.sync_copy(w_v, w_out.at[:, pl.ds(base, TOK_PER_W)])


_topk_call = pl.kernel(
    _topk_tec,
    out_type=[
        jax.ShapeDtypeStruct((TOPK, TOKENS), jnp.int32),
        jax.ShapeDtypeStruct((TOPK, TOKENS), jnp.float32),
    ],
    mesh=plsc.VectorSubcoreMesh(
        core_axis_name="c", subcore_axis_name="s",
        num_cores=NC, num_subcores=NS),
    compiler_params=pltpu.CompilerParams(needs_layout_passes=False),
    scratch_types=[
        pltpu.VMEM((EXPERTS, TOK_PER_W), jnp.float32),
        pltpu.VMEM((TOPK, TOK_PER_W), jnp.int32),
        pltpu.VMEM((TOPK, TOK_PER_W), jnp.float32),
    ],
)


def kernel(h, W):
    hf = h.reshape(TOKENS, DMODEL)
    lgT = _logits_t(hf, W)
    ids_sm, w_sm = _topk_call(lgT)
    return (ids_sm.T, w_sm.T, jnp.float32(0.0))
